# baseline (speedup 1.0000x reference)
import functools

import jax
import jax.numpy as jnp
from jax.experimental import pallas as pl
from jax.experimental.pallas import tpu as pltpu
from jax.experimental.pallas import tpu_sc as plsc

N = 100000
C = 100
D = 256
B = 16384
LAMBDA = 1.0

NW = 32
NPAD = 100352
SLICE = NPAD // NW
BPW = B // NW

NTC = 65536
NSTR = 2
RN = 2048
NBN = NTC // (RN * NSTR)
BPS = NTC // NSTR // RN
SROWS = 1088
SCH = 16
CPAD = 112

RB = 2048
NBB = B // RB
STEPS_BATCH = 2 * NBB


def _sc_pre_body(idx_hbm, labf_hbm, uf_hbm, of_hbm, labb_hbm, uraw_hbm,
                 idx_v, of_v, tab_v, gout_v, sem):
    wid = jax.lax.axis_index("s") * 2 + jax.lax.axis_index("c")
    base = wid * SLICE
    pltpu.sync_copy(idx_hbm, idx_v)

    def _zero(k, carry):
        of_v[pl.ds(k * 16, 16)] = jnp.zeros((16,), jnp.float32)
        return carry

    jax.lax.fori_loop(0, SLICE // 16, _zero, 0)
    ones16 = jnp.ones((16,), jnp.float32)

    def _scan(k, carry):
        for t in range(4):
            v = idx_v[pl.ds((4 * k + t) * 16, 16)]
            m = (v >= base) & (v < base + SLICE)
            plsc.store_scatter(of_v, [v - base], ones16, mask=m)
        return carry

    jax.lax.fori_loop(0, B // 64, _scan, 0)
    pltpu.sync_copy(of_v, of_hbm.at[pl.ds(base, SLICE)])

    bbase = wid * BPW

    def _gather(k, carry):
        vi = idx_v[pl.ds(bbase + k * 16, 16)]
        gout_v[pl.ds(k * 16, 16)] = plsc.load_gather(tab_v, [vi])
        return carry

    pltpu.sync_copy(uf_hbm, tab_v)
    jax.lax.fori_loop(0, BPW // 16, _gather, 0)
    pltpu.sync_copy(gout_v, uraw_hbm.at[pl.ds(bbase, BPW)])

    pltpu.sync_copy(labf_hbm, tab_v)
    jax.lax.fori_loop(0, BPW // 16, _gather, 0)
    pltpu.sync_copy(gout_v, labb_hbm.at[pl.ds(bbase, BPW)])


_sc_pre = functools.partial(
    pl.kernel,
    out_type=(jax.ShapeDtypeStruct((NPAD,), jnp.float32),
              jax.ShapeDtypeStruct((B,), jnp.float32),
              jax.ShapeDtypeStruct((B,), jnp.float32)),
    mesh=plsc.VectorSubcoreMesh(core_axis_name="c", subcore_axis_name="s"),
    scratch_types=[
        pltpu.VMEM((B,), jnp.int32),
        pltpu.VMEM((SLICE,), jnp.float32),
        pltpu.VMEM((N,), jnp.float32),
        pltpu.VMEM((BPW,), jnp.float32),
        pltpu.SemaphoreType.DMA,
    ],
    compiler_params=pltpu.CompilerParams(needs_layout_passes=False,
                                         use_tc_tiling_on_sc=False),
)(_sc_pre_body)


def _sc_sum_body(past_hbm, labi_hbm, of_hbm, zeros_hbm, part_hbm,
                 sums_v, lab_v, ofl_v, buf0, buf1, sem0, sem1):
    wid = jax.lax.axis_index("s") * 2 + jax.lax.axis_index("c")
    start = NTC + SROWS * wid
    start2 = jnp.minimum(start, N - SROWS)
    off0 = start - start2
    nrows = SROWS - off0
    nchunks = nrows // SCH

    pltpu.sync_copy(labi_hbm.at[pl.ds(start2, SROWS)], lab_v)
    pltpu.sync_copy(of_hbm.at[pl.ds(start2, SROWS)], ofl_v)
    pltpu.sync_copy(zeros_hbm, sums_v)

    def _row_slice(c):
        return past_hbm.at[pl.ds(start2 + off0 + c * SCH, SCH), :]

    cp0 = pltpu.async_copy(_row_slice(0), buf0, sem0)
    cp1 = pltpu.async_copy(_row_slice(1), buf1, sem1)
    del cp0, cp1

    def _consume(buf, c):
        g0 = off0 + c * SCH
        labv = lab_v[pl.ds(g0, 16)]
        wv = 1.0 - ofl_v[pl.ds(g0, 16)]
        for r in range(SCH):
            lab = labv[r]
            w = wv[r]
            base = lab * D
            for j in range(D // 16):
                sl = pl.ds(base + j * 16, 16)
                sums_v[sl] = sums_v[sl] + buf[r, pl.ds(j * 16, 16)] * w

    def _pair2(p, carry):
        c0 = 2 * p
        pltpu.make_async_copy(_row_slice(c0), buf0, sem0).wait()
        _consume(buf0, c0)

        @pl.when(c0 + 2 < nchunks)
        def _s0():
            pltpu.async_copy(_row_slice(c0 + 2), buf0, sem0)

        c1 = 2 * p + 1
        pltpu.make_async_copy(_row_slice(c1), buf1, sem1).wait()
        _consume(buf1, c1)

        @pl.when(c1 + 2 < nchunks)
        def _s1():
            pltpu.async_copy(_row_slice(c1 + 2), buf1, sem1)

        return carry

    jax.lax.fori_loop(0, nchunks // 2, _pair2, 0)

    pltpu.sync_copy(sums_v, part_hbm.at[pl.ds(wid * (C * D), C * D)])


_sc_sum = functools.partial(
    pl.kernel,
    out_type=jax.ShapeDtypeStruct((NW * C * D,), jnp.float32),
    mesh=plsc.VectorSubcoreMesh(core_axis_name="c", subcore_axis_name="s"),
    scratch_types=[
        pltpu.VMEM((C * D,), jnp.float32),
        pltpu.VMEM((SROWS,), jnp.int32),
        pltpu.VMEM((SROWS,), jnp.float32),
        pltpu.VMEM((SCH, D), jnp.float32),
        pltpu.VMEM((SCH, D), jnp.float32),
        pltpu.SemaphoreType.DMA,
        pltpu.SemaphoreType.DMA,
    ],
    compiler_params=pltpu.CompilerParams(needs_layout_passes=False,
                                         use_tc_tiling_on_sc=False),
)(_sc_sum_body)


def _tc_stream_body(*refs):
    past_refs = refs[0:NSTR]
    lab_refs = refs[NSTR:2 * NSTR]
    of_refs = refs[2 * NSTR:3 * NSTR]
    sums_ref, counts_ref = refs[3 * NSTR:]
    i = pl.program_id(0)
    iota_col = jax.lax.broadcasted_iota(jnp.int32, (C, 1), 0)

    @pl.when(i == 0)
    def _init():
        sums_ref[...] = jnp.zeros_like(sums_ref)
        counts_ref[...] = jnp.zeros_like(counts_ref)

    for k in range(NSTR):
        past = past_refs[k][...]
        labels = lab_refs[k][0]
        o = of_refs[k][0]
        oh_t = (labels == iota_col).astype(jnp.float32)
        counts_ref[...] += jnp.sum(oh_t, axis=1, keepdims=True)
        ohm_t = (oh_t * (1.0 - o)).astype(jnp.bfloat16)
        sums_ref[...] += jax.lax.dot_general(
            ohm_t, past.astype(jnp.bfloat16), (((1,), (0,)), ((), ())),
            preferred_element_type=jnp.float32)


def _idx_past(k):
    return lambda i: (k * BPS + i, 0)


def _idx_rows_n(k):
    return lambda i: (k * BPS + i, 0, 0)


@jax.jit
def _tc_stream(past, labels3, oflags3):
    return pl.pallas_call(
        _tc_stream_body,
        grid=(NBN,),
        in_specs=[
            *[pl.BlockSpec((RN, D), _idx_past(k)) for k in range(NSTR)],
            *[pl.BlockSpec((1, 1, RN), _idx_rows_n(k)) for k in range(NSTR)],
            *[pl.BlockSpec((1, 1, RN), _idx_rows_n(k)) for k in range(NSTR)],
        ],
        out_specs=[pl.BlockSpec((C, D), lambda i: (0, 0)),
                   pl.BlockSpec((C, 1), lambda i: (0, 0))],
        out_shape=[jax.ShapeDtypeStruct((C, D), jnp.float32),
                   jax.ShapeDtypeStruct((C, 1), jnp.float32)],
        compiler_params=pltpu.CompilerParams(
            dimension_semantics=("arbitrary",)),
    )(*([past] * NSTR), *([labels3] * NSTR), *([oflags3] * NSTR))


def _tc_batch_body(sumss_ref, countss_ref, part_ref, tail_ref, emb_ref,
                   labb_ref, logits_ref, targets_ref, uraw_ref, centroids_ref,
                   out_ref, sums_ref, counts_ref, centnt_ref, embn_ref,
                   acc_ref):
    i = pl.program_id(0)
    iota_col = jax.lax.broadcasted_iota(jnp.int32, (C, 1), 0)

    @pl.when(i == 0)
    def _init():
        sums_ref[...] = sumss_ref[...] + jnp.sum(
            part_ref[...].reshape(NW, C, D), axis=0)
        cnt = countss_ref[...]
        for t in range(4):
            lt = tail_ref[:, pl.ds(t * ((N - NTC) // 4), (N - NTC) // 4)]
            ohc = (lt == iota_col).astype(jnp.float32)
            cnt += jnp.sum(ohc, axis=1, keepdims=True)
        counts_ref[...] = cnt
        acc_ref[0] = 0.0
        acc_ref[1] = 0.0
        acc_ref[2] = 0.0

    @pl.when(i < NBB)
    def _corr():
        e = emb_ref[...]
        ss = jnp.sum(e * e, axis=1, keepdims=True)
        emb = e * (1.0 / jnp.maximum(jnp.sqrt(ss), 1e-12))
        embn_ref[pl.ds(i * RB, RB), :] = emb
        labb = labb_ref[0]
        ohb_t = (labb == iota_col).astype(jnp.bfloat16)
        sums_ref[...] += jax.lax.dot_general(
            ohb_t, emb.astype(jnp.bfloat16), (((1,), (0,)), ((), ())),
            preferred_element_type=jnp.float32)

    @pl.when(i == NBB - 1)
    def _finalize():
        sums = sums_ref[...]
        counts = counts_ref[...]
        means = sums / jnp.maximum(counts, 1.0)
        cent = jnp.where(counts > 0, means, centroids_ref[...])
        nrm = jnp.sqrt(jnp.sum(cent * cent, axis=1, keepdims=True))
        centn = cent / jnp.maximum(nrm, 1e-12)
        centnt_ref[...] = centn.T

    @pl.when(i >= NBB)
    def _loss():
        j = i - NBB
        iota_row = jax.lax.broadcasted_iota(jnp.int32, (1, C), 1)
        emb = embn_ref[pl.ds(j * RB, RB), :]
        logits = logits_ref[...]
        sl_logits = jax.lax.dot_general(
            emb.astype(jnp.bfloat16), centnt_ref[...].astype(jnp.bfloat16),
            (((1,), (0,)), ((), ())),
            preferred_element_type=jnp.float32)
        ex = jnp.exp(sl_logits)
        soft = ex * (1.0 / jnp.sum(ex, axis=1, keepdims=True))
        el = jnp.exp(logits)
        sel = jnp.sum(el, axis=1, keepdims=True)
        probs = el * (1.0 / sel)
        u_v = 1.0 / (1.0 + jnp.exp(-uraw_ref[0]))
        anum = jnp.maximum(probs + u_v * soft, 1e-6)
        asum = jnp.sum(anum, axis=1, keepdims=True)
        adjusted = anum * (1.0 / asum)
        oht = (targets_ref[0] == iota_row).astype(jnp.float32)
        tgt_logit = jnp.sum(oht * logits, axis=1, keepdims=True)
        ce = jnp.log(sel) - tgt_logit
        acc_ref[0] += jnp.sum((1.0 - u_v) * ce)
        acc_ref[1] += jnp.sum(jnp.log(asum)) - jnp.sum(soft * jnp.log(anum))
        acc_ref[2] += jnp.sum((adjusted - soft) ** 2)

    @pl.when(i == STEPS_BATCH - 1)
    def _out():
        loss = (acc_ref[0] + acc_ref[1]) / B + LAMBDA * acc_ref[2] / (B * C)
        out_ref[...] = jnp.broadcast_to(loss, (1, 1))


def _idx_emb(i):
    return (jnp.clip(i, 0, NBB - 1), 0)


def _idx_labb(i):
    return (jnp.clip(i, 0, NBB - 1), 0, 0)


def _idx_logits(i):
    return (jnp.clip(i - NBB, 0, NBB - 1), 0)


def _idx_rows_b(i):
    return (jnp.clip(i - NBB, 0, NBB - 1), 0, 0)


@jax.jit
def _tc_batch(sums_s, counts_s, partials, labels_tail, embeddings, labb3,
              logits, targets3, uraw3, centroids):
    out = pl.pallas_call(
        _tc_batch_body,
        grid=(STEPS_BATCH,),
        in_specs=[
            pl.BlockSpec((C, D), lambda i: (0, 0)),
            pl.BlockSpec((C, 1), lambda i: (0, 0)),
            pl.BlockSpec((NW * C, D), lambda i: (0, 0)),
            pl.BlockSpec((1, N - NTC), lambda i: (0, 0)),
            pl.BlockSpec((RB, D), _idx_emb),
            pl.BlockSpec((1, 1, RB), _idx_labb),
            pl.BlockSpec((RB, C), _idx_logits),
            pl.BlockSpec((1, RB, 1), _idx_rows_b),
            pl.BlockSpec((1, RB, 1), _idx_rows_b),
            pl.BlockSpec((C, D), lambda i: (0, 0)),
        ],
        out_specs=pl.BlockSpec((1, 1), lambda i: (0, 0)),
        out_shape=jax.ShapeDtypeStruct((1, 1), jnp.float32),
        scratch_shapes=[
            pltpu.VMEM((C, D), jnp.float32),
            pltpu.VMEM((C, 1), jnp.float32),
            pltpu.VMEM((D, C), jnp.float32),
            pltpu.VMEM((B, D), jnp.float32),
            pltpu.SMEM((4,), jnp.float32),
        ],
        compiler_params=pltpu.CompilerParams(
            dimension_semantics=("arbitrary",)),
    )(sums_s, counts_s, partials, labels_tail, embeddings, labb3, logits,
      targets3, uraw3, centroids)
    return out[0, 0]


def kernel(logits, indexes, embeddings, targets, epoch, u, past_embeddings,
           centroids, labels):
    idx = indexes.astype(jnp.int32)
    labels_i = labels.astype(jnp.int32)
    of_pad, labb_f, u_raw = _sc_pre(idx, labels_i.astype(jnp.float32),
                                    u[:, 0])
    lab_b = labb_f.astype(jnp.int32)
    zeros_cd = jnp.zeros((C * D,), jnp.float32)
    part_flat = _sc_sum(past_embeddings, labels_i, of_pad, zeros_cd)
    partials = part_flat.reshape(NW * C, D)
    labels3 = labels_i[:NTC].reshape(NTC // RN, 1, RN)
    oflags3 = of_pad[:NTC].reshape(NTC // RN, 1, RN)
    sums_s, counts_s = _tc_stream(past_embeddings, labels3, oflags3)
    labb3 = lab_b.reshape(NBB, 1, RB)
    targets3 = targets.astype(jnp.int32).reshape(NBB, RB, 1)
    uraw3 = u_raw.reshape(NBB, RB, 1)
    labels_tail = labels_i[NTC:].reshape(1, N - NTC)
    return _tc_batch(sums_s, counts_s, partials, labels_tail, embeddings,
                     labb3, logits, targets3, uraw3, centroids)


# speedup vs baseline: 2.5238x; 2.5238x over previous
import functools

import jax
import jax.numpy as jnp
from jax.experimental import pallas as pl
from jax.experimental.pallas import tpu as pltpu
from jax.experimental.pallas import tpu_sc as plsc

N = 100000
C = 100
D = 256
B = 16384
LAMBDA = 1.0

NW = 32
NPAD = 100352
SLICE = NPAD // NW
BPW = B // NW
GCH = 128
NCH = BPW // GCH

NSTR = 2
RN = 5000
NBN = N // (RN * NSTR)
BPS = N // NSTR // RN
RB = 2048
NBB = B // RB
STEPS = NBN + 2 * NBB


def _sc_body(idx_hbm, labf_hbm, uf_hbm, of_hbm, labb_hbm, uraw_hbm,
             idx_v, of_v, tab_v, gout_v, sem):
    wid = jax.lax.axis_index("s") * 2 + jax.lax.axis_index("c")
    base = wid * SLICE
    pltpu.sync_copy(idx_hbm, idx_v)

    def _zero(k, carry):
        of_v[pl.ds(k * 16, 16)] = jnp.zeros((16,), jnp.float32)
        return carry

    jax.lax.fori_loop(0, SLICE // 16, _zero, 0)
    ones16 = jnp.ones((16,), jnp.float32)

    def _scan(k, carry):
        for t in range(4):
            v = idx_v[pl.ds((4 * k + t) * 16, 16)]
            m = (v >= base) & (v < base + SLICE)
            plsc.store_scatter(of_v, [v - base], ones16, mask=m)
        return carry

    jax.lax.fori_loop(0, B // 64, _scan, 0)
    pltpu.sync_copy(of_v, of_hbm.at[pl.ds(base, SLICE)])

    bbase = wid * BPW

    def _gather(k, carry):
        vi = idx_v[pl.ds(bbase + k * 16, 16)]
        gout_v[pl.ds(k * 16, 16)] = plsc.load_gather(tab_v, [vi])
        return carry

    pltpu.sync_copy(uf_hbm, tab_v)
    jax.lax.fori_loop(0, BPW // 16, _gather, 0)
    pltpu.sync_copy(gout_v, uraw_hbm.at[pl.ds(bbase, BPW)])

    pltpu.sync_copy(labf_hbm, tab_v)
    jax.lax.fori_loop(0, BPW // 16, _gather, 0)
    pltpu.sync_copy(gout_v, labb_hbm.at[pl.ds(bbase, BPW)])


_sc_pre = functools.partial(
    pl.kernel,
    out_type=(jax.ShapeDtypeStruct((NPAD,), jnp.float32),
              jax.ShapeDtypeStruct((B,), jnp.float32),
              jax.ShapeDtypeStruct((B,), jnp.float32)),
    mesh=plsc.VectorSubcoreMesh(core_axis_name="c", subcore_axis_name="s"),
    scratch_types=[
        pltpu.VMEM((B,), jnp.int32),
        pltpu.VMEM((SLICE,), jnp.float32),
        pltpu.VMEM((N,), jnp.float32),
        pltpu.VMEM((BPW,), jnp.float32),
        pltpu.SemaphoreType.DMA,
    ],
    compiler_params=pltpu.CompilerParams(needs_layout_passes=False,
                                         use_tc_tiling_on_sc=False),
)(_sc_body)


def _tc_body(*refs):
    past_refs = refs[0:NSTR]
    lab_refs = refs[NSTR:2 * NSTR]
    of_refs = refs[2 * NSTR:3 * NSTR]
    (emb_ref, labb_ref, logits_ref, targets_ref, uraw_ref, centroids_ref,
     out_ref, sums_ref, counts_ref, centnt_ref, embn_ref, acc_ref) = \
        refs[3 * NSTR:]
    i = pl.program_id(0)
    iota_col = jax.lax.broadcasted_iota(jnp.int32, (C, 1), 0)

    @pl.when(i == 0)
    def _init():
        sums_ref[...] = jnp.zeros_like(sums_ref)
        counts_ref[...] = jnp.zeros_like(counts_ref)
        acc_ref[0] = 0.0
        acc_ref[1] = 0.0
        acc_ref[2] = 0.0

    @pl.when(i < NBN)
    def _stream():
        for k in range(NSTR):
            past = past_refs[k][...]
            labels = lab_refs[k][0]
            o = of_refs[k][0]
            oh_t = (labels == iota_col).astype(jnp.float32)
            counts_ref[...] += jnp.sum(oh_t, axis=1, keepdims=True)
            ohm_t = (oh_t * (1.0 - o)).astype(jnp.bfloat16)
            sums_ref[...] += jax.lax.dot_general(
                ohm_t, past.astype(jnp.bfloat16), (((1,), (0,)), ((), ())),
                preferred_element_type=jnp.float32)

    @pl.when((i >= NBN) & (i < NBN + NBB))
    def _corr():
        j = i - NBN
        e = emb_ref[...]
        ss = jnp.sum(e * e, axis=1, keepdims=True)
        emb = e * (1.0 / jnp.maximum(jnp.sqrt(ss), 1e-12))
        embn_ref[pl.ds(j * RB, RB), :] = emb
        labb = labb_ref[0]
        ohb_t = (labb == iota_col).astype(jnp.bfloat16)
        sums_ref[...] += jax.lax.dot_general(
            ohb_t, emb.astype(jnp.bfloat16), (((1,), (0,)), ((), ())),
            preferred_element_type=jnp.float32)

    @pl.when(i == NBN + NBB - 1)
    def _finalize():
        sums = sums_ref[...]
        counts = counts_ref[...]
        means = sums / jnp.maximum(counts, 1.0)
        cent = jnp.where(counts > 0, means, centroids_ref[...])
        nrm = jnp.sqrt(jnp.sum(cent * cent, axis=1, keepdims=True))
        centn = cent / jnp.maximum(nrm, 1e-12)
        centnt_ref[...] = centn.T

    @pl.when(i >= NBN + NBB)
    def _loss():
        j = i - NBN - NBB
        iota_row = jax.lax.broadcasted_iota(jnp.int32, (1, C), 1)
        emb = embn_ref[pl.ds(j * RB, RB), :]
        logits = logits_ref[...]
        sl_logits = jax.lax.dot_general(
            emb.astype(jnp.bfloat16), centnt_ref[...].astype(jnp.bfloat16),
            (((1,), (0,)), ((), ())),
            preferred_element_type=jnp.float32)
        ex = jnp.exp(sl_logits)
        soft = ex * (1.0 / jnp.sum(ex, axis=1, keepdims=True))
        el = jnp.exp(logits)
        sel = jnp.sum(el, axis=1, keepdims=True)
        probs = el * (1.0 / sel)
        u_v = 1.0 / (1.0 + jnp.exp(-uraw_ref[0]))
        anum = jnp.maximum(probs + u_v * soft, 1e-6)
        asum = jnp.sum(anum, axis=1, keepdims=True)
        adjusted = anum * (1.0 / asum)
        oht = (targets_ref[0] == iota_row).astype(jnp.float32)
        tgt_logit = jnp.sum(oht * logits, axis=1, keepdims=True)
        ce = jnp.log(sel) - tgt_logit
        acc_ref[0] += jnp.sum((1.0 - u_v) * ce)
        acc_ref[1] += jnp.sum(jnp.log(asum)) - jnp.sum(soft * jnp.log(anum))
        acc_ref[2] += jnp.sum((adjusted - soft) ** 2)

    @pl.when(i == STEPS - 1)
    def _out():
        loss = (acc_ref[0] + acc_ref[1]) / B + LAMBDA * acc_ref[2] / (B * C)
        out_ref[...] = jnp.broadcast_to(loss, (1, 1))


def _idx_past(k):
    return lambda i: (k * BPS + jnp.minimum(i, NBN - 1), 0)


def _idx_rows_n(k):
    return lambda i: (k * BPS + jnp.minimum(i, NBN - 1), 0, 0)


def _idx_emb(i):
    return (jnp.clip(i - NBN, 0, NBB - 1), 0)


def _idx_labb(i):
    return (jnp.clip(i - NBN, 0, NBB - 1), 0, 0)


def _idx_logits(i):
    return (jnp.clip(i - NBN - NBB, 0, NBB - 1), 0)


def _idx_rows_b(i):
    return (jnp.clip(i - NBN - NBB, 0, NBB - 1), 0, 0)


@functools.partial(jax.jit, static_argnames=("interpret",))
def _tc_call(past, labels3, oflags3, embeddings, labb3, logits, targets3,
             uraw3, centroids, interpret=False):
    out = pl.pallas_call(
        _tc_body,
        grid=(STEPS,),
        in_specs=[
            *[pl.BlockSpec((RN, D), _idx_past(k)) for k in range(NSTR)],
            *[pl.BlockSpec((1, 1, RN), _idx_rows_n(k)) for k in range(NSTR)],
            *[pl.BlockSpec((1, 1, RN), _idx_rows_n(k)) for k in range(NSTR)],
            pl.BlockSpec((RB, D), _idx_emb),
            pl.BlockSpec((1, 1, RB), _idx_labb),
            pl.BlockSpec((RB, C), _idx_logits),
            pl.BlockSpec((1, RB, 1), _idx_rows_b),
            pl.BlockSpec((1, RB, 1), _idx_rows_b),
            pl.BlockSpec((C, D), lambda i: (0, 0)),
        ],
        out_specs=pl.BlockSpec((1, 1), lambda i: (0, 0)),
        out_shape=jax.ShapeDtypeStruct((1, 1), jnp.float32),
        scratch_shapes=[
            pltpu.VMEM((C, D), jnp.float32),
            pltpu.VMEM((C, 1), jnp.float32),
            pltpu.VMEM((D, C), jnp.float32),
            pltpu.VMEM((B, D), jnp.float32),
            pltpu.SMEM((4,), jnp.float32),
        ],
        compiler_params=pltpu.CompilerParams(
            dimension_semantics=("arbitrary",)),
        interpret=interpret,
    )(*([past] * NSTR), *([labels3] * NSTR), *([oflags3] * NSTR),
      embeddings, labb3, logits, targets3, uraw3, centroids)
    return out[0, 0]


def kernel(logits, indexes, embeddings, targets, epoch, u, past_embeddings,
           centroids, labels):
    idx = indexes.astype(jnp.int32)
    labels_i = labels.astype(jnp.int32)
    of_pad, labb_f, u_raw = _sc_pre(idx, labels_i.astype(jnp.float32),
                                    u[:, 0])
    oflags = of_pad[:N]
    lab_b = labb_f.astype(jnp.int32)
    labels3 = labels_i.reshape(N // RN, 1, RN)
    oflags3 = oflags.reshape(N // RN, 1, RN)
    labb3 = lab_b.reshape(NBB, 1, RB)
    targets3 = targets.astype(jnp.int32).reshape(NBB, RB, 1)
    uraw3 = u_raw.reshape(NBB, RB, 1)
    return _tc_call(past_embeddings, labels3, oflags3, embeddings, labb3,
                    logits, targets3, uraw3, centroids)


# speedup vs baseline: 2.6987x; 1.0693x over previous
import functools

import jax
import jax.numpy as jnp
from jax.experimental import pallas as pl
from jax.experimental.pallas import tpu as pltpu
from jax.experimental.pallas import tpu_sc as plsc

N = 100000
C = 100
D = 256
B = 16384
LAMBDA = 1.0

NW = 32
NPAD = 100352
SLICE = NPAD // NW
BPW = B // NW
BPW2 = B // 16

NSTR = 2
RN = 5000
NBN = N // (RN * NSTR)
BPS = N // NSTR // RN
RB = 2048
NBB = B // RB
STEPS = NBN + 2 * NBB


def _sc_body(idx_hbm, labf_hbm, uf_hbm, zero_hbm, of_hbm, labb_hbm, uraw_hbm,
             idx_v, of_v, tab_v, gout_v, sem):
    wid = jax.lax.axis_index("s") * 2 + jax.lax.axis_index("c")
    base = wid * SLICE
    pltpu.sync_copy(idx_hbm, idx_v)

    pltpu.sync_copy(zero_hbm, of_v)
    ones16 = jnp.ones((16,), jnp.float32)

    def _scan(k, carry):
        for t in range(4):
            v = idx_v[pl.ds((4 * k + t) * 16, 16)]
            m = (v >= base) & (v < base + SLICE)
            plsc.store_scatter(of_v, [v - base], ones16, mask=m)
        return carry

    jax.lax.fori_loop(0, B // 64, _scan, 0)
    pltpu.sync_copy(of_v, of_hbm.at[pl.ds(base, SLICE)])

    half = wid < 16
    bbase = jnp.where(half, wid * BPW2, (wid - 16) * BPW2)

    def _gather(k, carry):
        vi = idx_v[pl.ds(bbase + k * 16, 16)]
        gout_v[pl.ds(k * 16, 16)] = plsc.load_gather(tab_v, [vi])
        return carry

    @pl.when(half)
    def _do_u():
        pltpu.sync_copy(uf_hbm, tab_v)
        jax.lax.fori_loop(0, BPW2 // 16, _gather, 0)
        pltpu.sync_copy(gout_v, uraw_hbm.at[pl.ds(bbase, BPW2)])

    @pl.when(jnp.logical_not(half))
    def _do_lab():
        pltpu.sync_copy(labf_hbm, tab_v)
        jax.lax.fori_loop(0, BPW2 // 16, _gather, 0)
        pltpu.sync_copy(gout_v, labb_hbm.at[pl.ds(bbase, BPW2)])


_sc_pre = functools.partial(
    pl.kernel,
    out_type=(jax.ShapeDtypeStruct((NPAD,), jnp.float32),
              jax.ShapeDtypeStruct((B,), jnp.float32),
              jax.ShapeDtypeStruct((B,), jnp.float32)),
    mesh=plsc.VectorSubcoreMesh(core_axis_name="c", subcore_axis_name="s"),
    scratch_types=[
        pltpu.VMEM((B,), jnp.int32),
        pltpu.VMEM((SLICE,), jnp.float32),
        pltpu.VMEM((N,), jnp.float32),
        pltpu.VMEM((BPW2,), jnp.float32),
        pltpu.SemaphoreType.DMA,
    ],
    compiler_params=pltpu.CompilerParams(needs_layout_passes=False,
                                         use_tc_tiling_on_sc=False),
)(_sc_body)


def _tc_body(*refs):
    past_refs = refs[0:NSTR]
    lab_refs = refs[NSTR:2 * NSTR]
    of_refs = refs[2 * NSTR:3 * NSTR]
    (emb_ref, labb_ref, logits_ref, targets_ref, uraw_ref, centroids_ref,
     out_ref, sums_ref, counts_ref, centnt_ref, embn_ref, acc_ref) = \
        refs[3 * NSTR:]
    i = pl.program_id(0)
    iota_col = jax.lax.broadcasted_iota(jnp.int32, (C, 1), 0)

    @pl.when(i == 0)
    def _init():
        sums_ref[...] = jnp.zeros_like(sums_ref)
        counts_ref[...] = jnp.zeros_like(counts_ref)
        acc_ref[0] = 0.0
        acc_ref[1] = 0.0
        acc_ref[2] = 0.0

    @pl.when(i < NBN)
    def _stream():
        for k in range(NSTR):
            past = past_refs[k][...]
            labels = lab_refs[k][0]
            o = of_refs[k][0]
            oh_t = (labels == iota_col).astype(jnp.float32)
            counts_ref[...] += jnp.sum(oh_t, axis=1, keepdims=True)
            ohm_t = (oh_t * (1.0 - o)).astype(jnp.bfloat16)
            sums_ref[...] += jax.lax.dot_general(
                ohm_t, past.astype(jnp.bfloat16), (((1,), (0,)), ((), ())),
                preferred_element_type=jnp.float32)

    @pl.when((i >= NBN) & (i < NBN + NBB))
    def _corr():
        j = i - NBN
        e = emb_ref[...]
        ss = jnp.sum(e * e, axis=1, keepdims=True)
        emb = e * (1.0 / jnp.maximum(jnp.sqrt(ss), 1e-12))
        embn_ref[pl.ds(j * RB, RB), :] = emb
        labb = labb_ref[0]
        ohb_t = (labb == iota_col).astype(jnp.bfloat16)
        sums_ref[...] += jax.lax.dot_general(
            ohb_t, emb.astype(jnp.bfloat16), (((1,), (0,)), ((), ())),
            preferred_element_type=jnp.float32)

    @pl.when(i == NBN + NBB - 1)
    def _finalize():
        sums = sums_ref[...]
        counts = counts_ref[...]
        means = sums / jnp.maximum(counts, 1.0)
        cent = jnp.where(counts > 0, means, centroids_ref[...])
        nrm = jnp.sqrt(jnp.sum(cent * cent, axis=1, keepdims=True))
        centn = cent / jnp.maximum(nrm, 1e-12)
        centnt_ref[...] = centn.T

    @pl.when(i >= NBN + NBB)
    def _loss():
        j = i - NBN - NBB
        iota_row = jax.lax.broadcasted_iota(jnp.int32, (1, C), 1)
        emb = embn_ref[pl.ds(j * RB, RB), :]
        logits = logits_ref[...]
        sl_logits = jax.lax.dot_general(
            emb.astype(jnp.bfloat16), centnt_ref[...].astype(jnp.bfloat16),
            (((1,), (0,)), ((), ())),
            preferred_element_type=jnp.float32)
        ex = jnp.exp(sl_logits)
        soft = ex * (1.0 / jnp.sum(ex, axis=1, keepdims=True))
        el = jnp.exp(logits)
        sel = jnp.sum(el, axis=1, keepdims=True)
        probs = el * (1.0 / sel)
        u_v = 1.0 / (1.0 + jnp.exp(-uraw_ref[0]))
        anum = jnp.maximum(probs + u_v * soft, 1e-6)
        asum = jnp.sum(anum, axis=1, keepdims=True)
        adjusted = anum * (1.0 / asum)
        oht = (targets_ref[0] == iota_row).astype(jnp.float32)
        tgt_logit = jnp.sum(oht * logits, axis=1, keepdims=True)
        ce = jnp.log(sel) - tgt_logit
        acc_ref[0] += jnp.sum((1.0 - u_v) * ce)
        acc_ref[1] += jnp.sum(jnp.log(asum)) - jnp.sum(soft * jnp.log(anum))
        acc_ref[2] += jnp.sum((adjusted - soft) ** 2)

    @pl.when(i == STEPS - 1)
    def _out():
        loss = (acc_ref[0] + acc_ref[1]) / B + LAMBDA * acc_ref[2] / (B * C)
        out_ref[...] = jnp.broadcast_to(loss, (1, 1))


def _idx_past(k):
    return lambda i: (k * BPS + jnp.minimum(i, NBN - 1), 0)


def _idx_rows_n(k):
    return lambda i: (k * BPS + jnp.minimum(i, NBN - 1), 0, 0)


def _idx_emb(i):
    return (jnp.clip(i - NBN, 0, NBB - 1), 0)


def _idx_labb(i):
    return (jnp.clip(i - NBN, 0, NBB - 1), 0, 0)


def _idx_logits(i):
    return (jnp.clip(i - NBN - NBB, 0, NBB - 1), 0)


def _idx_rows_b(i):
    return (jnp.clip(i - NBN - NBB, 0, NBB - 1), 0, 0)


@functools.partial(jax.jit, static_argnames=("interpret",))
def _tc_call(past, labels3, oflags3, embeddings, labb3, logits, targets3,
             uraw3, centroids, interpret=False):
    out = pl.pallas_call(
        _tc_body,
        grid=(STEPS,),
        in_specs=[
            *[pl.BlockSpec((RN, D), _idx_past(k)) for k in range(NSTR)],
            *[pl.BlockSpec((1, 1, RN), _idx_rows_n(k)) for k in range(NSTR)],
            *[pl.BlockSpec((1, 1, RN), _idx_rows_n(k)) for k in range(NSTR)],
            pl.BlockSpec((RB, D), _idx_emb),
            pl.BlockSpec((1, 1, RB), _idx_labb),
            pl.BlockSpec((RB, C), _idx_logits),
            pl.BlockSpec((1, RB, 1), _idx_rows_b),
            pl.BlockSpec((1, RB, 1), _idx_rows_b),
            pl.BlockSpec((C, D), lambda i: (0, 0)),
        ],
        out_specs=pl.BlockSpec((1, 1), lambda i: (0, 0)),
        out_shape=jax.ShapeDtypeStruct((1, 1), jnp.float32),
        scratch_shapes=[
            pltpu.VMEM((C, D), jnp.float32),
            pltpu.VMEM((C, 1), jnp.float32),
            pltpu.VMEM((D, C), jnp.float32),
            pltpu.VMEM((B, D), jnp.float32),
            pltpu.SMEM((4,), jnp.float32),
        ],
        compiler_params=pltpu.CompilerParams(
            dimension_semantics=("arbitrary",)),
        interpret=interpret,
    )(*([past] * NSTR), *([labels3] * NSTR), *([oflags3] * NSTR),
      embeddings, labb3, logits, targets3, uraw3, centroids)
    return out[0, 0]


def kernel(logits, indexes, embeddings, targets, epoch, u, past_embeddings,
           centroids, labels):
    idx = indexes.astype(jnp.int32)
    labels_i = labels.astype(jnp.int32)
    of_pad, labb_f, u_raw = _sc_pre(idx, labels_i.astype(jnp.float32),
                                    u[:, 0], jnp.zeros((SLICE,), jnp.float32))
    oflags = of_pad[:N]
    lab_b = labb_f.astype(jnp.int32)
    labels3 = labels_i.reshape(N // RN, 1, RN)
    oflags3 = oflags.reshape(N // RN, 1, RN)
    labb3 = lab_b.reshape(NBB, 1, RB)
    targets3 = targets.astype(jnp.int32).reshape(NBB, RB, 1)
    uraw3 = u_raw.reshape(NBB, RB, 1)
    return _tc_call(past_embeddings, labels3, oflags3, embeddings, labb3,
                    logits, targets3, uraw3, centroids)
